# baseline (speedup 1.0000x reference)
import functools

import jax
import jax.numpy as jnp
from jax import lax
from jax.experimental import pallas as pl
from jax.experimental.pallas import tpu as pltpu
from jax.experimental.pallas import tpu_sc as plsc

B, C, H, W, I = 8, 16, 512, 512, 4096
NC, NS, L = 2, 16, 16
NW = NC * NS
BC = B * C
BC_PER_W = BC // NW
HW = H * W
CH = I
GROUPS = CH // L


def _loss_body(pred_hbm, iv0_hbm, iv1_hbm, nc0_hbm, nc1_hbm, out_hbm,
               ivbuf, bidx, didx, bvals, dvals, nc0buf, nc1buf, accbuf,
               sem_b, sem_d):
    cid = lax.axis_index("c")
    sid = lax.axis_index("s")
    wid = sid * NC + cid

    iota = lax.iota(jnp.int32, L)

    pltpu.sync_copy(nc0_hbm, nc0buf)
    pltpu.sync_copy(nc1_hbm, nc1buf)

    def index_body(j, carry):
        base_w = j * (4 * L)
        gx = base_w + iota * 4
        bx = plsc.load_gather(ivbuf, [gx])
        by = plsc.load_gather(ivbuf, [gx + 1])
        dx = plsc.load_gather(ivbuf, [gx + 2])
        dy = plsc.load_gather(ivbuf, [gx + 3])
        pbase = carry
        bidx[pl.ds(j * L, L)] = pbase + bx * W + by
        didx[pl.ds(j * L, L)] = pbase + dx * W + dy
        return carry

    acc = jnp.zeros((L,), jnp.float32)
    for iv_hbm, ncbuf in ((iv0_hbm, nc0buf), (iv1_hbm, nc1buf)):
        for k in range(BC_PER_W):
            bc = wid * BC_PER_W + k
            c = lax.rem(bc, C)
            t_vec = plsc.load_gather(ncbuf, [jnp.full((L,), c, jnp.int32)])
            t_vec = jnp.minimum(t_vec, I)

            pltpu.sync_copy(iv_hbm.at[pl.ds(bc * (CH * 4), CH * 4)], ivbuf)
            lax.fori_loop(0, GROUPS, index_body, bc * HW)

            cp_b = pltpu.async_copy(pred_hbm.at[bidx], bvals, sem_b)
            cp_d = pltpu.async_copy(pred_hbm.at[didx], dvals, sem_d)
            cp_b.wait()
            cp_d.wait()

            def acc_body(j, a):
                bv = bvals[pl.ds(j * L, L)]
                dv = dvals[pl.ds(j * L, L)]
                d = bv - dv
                d2 = d * d
                good = (j * L + iota) < t_vec
                return a + jnp.where(good, 1.0 - d2, d2)

            acc = lax.fori_loop(0, GROUPS, acc_body, acc)

    accbuf[...] = acc
    pltpu.sync_copy(accbuf, out_hbm.at[wid])


@jax.jit
def _loss(pred_flat, iv0_flat, iv1_flat, nc0, nc1):
    mesh = plsc.VectorSubcoreMesh(core_axis_name="c", subcore_axis_name="s")
    run = pl.kernel(
        _loss_body,
        out_type=jax.ShapeDtypeStruct((NW, L), jnp.float32),
        mesh=mesh,
        compiler_params=pltpu.CompilerParams(needs_layout_passes=False),
        scratch_types=[
            pltpu.VMEM((CH * 4,), jnp.int32),
            pltpu.VMEM((CH,), jnp.int32),
            pltpu.VMEM((CH,), jnp.int32),
            pltpu.VMEM((CH,), jnp.float32),
            pltpu.VMEM((CH,), jnp.float32),
            pltpu.VMEM((L,), jnp.int32),
            pltpu.VMEM((L,), jnp.int32),
            pltpu.VMEM((L,), jnp.float32),
            pltpu.SemaphoreType.DMA,
            pltpu.SemaphoreType.DMA,
        ],
    )
    partials = run(pred_flat, iv0_flat, iv1_flat, nc0, nc1)
    return jnp.sum(partials)


def kernel(prediction, intervals_comp_0, intervals_comp_1,
           good_intervals_0, good_intervals_1):
    pred_flat = prediction.reshape(-1)
    iv0_flat = intervals_comp_0.reshape(-1)
    iv1_flat = intervals_comp_1.reshape(-1)
    return _loss(pred_flat, iv0_flat, iv1_flat,
                 good_intervals_0, good_intervals_1)


# speedup vs baseline: 26.3165x; 26.3165x over previous
import functools

import jax
import jax.numpy as jnp
from jax import lax
from jax.experimental import pallas as pl
from jax.experimental.pallas import tpu as pltpu
from jax.experimental.pallas import tpu_sc as plsc

B, C, H, W, I = 8, 16, 512, 512, 4096
NC, NS, L = 2, 16, 16
NW = NC * NS
BC = B * C
BC_PER_W = BC // NW
HW = H * W
CH = I
GROUPS = CH // L


def _phys_addr(pbase, x, y):
    xhi = lax.shift_right_logical(x, 3)
    xlo = lax.bitwise_and(x, 7)
    yhi = lax.shift_right_logical(y, 7)
    ylo = lax.bitwise_and(y, 127)
    return pbase + xhi * 4096 + yhi * 1024 + xlo * 128 + ylo


def _loss_body(pred_hbm, iv0_hbm, iv1_hbm, nc0_hbm, nc1_hbm, out_hbm,
               bxbuf, bybuf, dxbuf, dybuf, bidx, didx, bvals, dvals,
               nc0buf, nc1buf, accbuf, sem_b, sem_d):
    cid = lax.axis_index("c")
    sid = lax.axis_index("s")
    wid = sid * NC + cid

    iota = lax.iota(jnp.int32, L)

    pltpu.sync_copy(nc0_hbm, nc0buf)
    pltpu.sync_copy(nc1_hbm, nc1buf)

    def index_body(j, pbase):
        sl = pl.ds(j * L, L)
        bidx[sl] = _phys_addr(pbase, bxbuf[sl], bybuf[sl])
        didx[sl] = _phys_addr(pbase, dxbuf[sl], dybuf[sl])
        return pbase

    acc = jnp.zeros((L,), jnp.float32)
    for iv_hbm, ncbuf in ((iv0_hbm, nc0buf), (iv1_hbm, nc1buf)):
        for k in range(BC_PER_W):
            bc = wid * BC_PER_W + k
            b = lax.div(bc, C)
            c = lax.rem(bc, C)
            t_vec = plsc.load_gather(ncbuf, [jnp.full((L,), c, jnp.int32)])
            t_vec = jnp.minimum(t_vec, I)

            pltpu.sync_copy(iv_hbm.at[b, c, 0, 0], bxbuf)
            pltpu.sync_copy(iv_hbm.at[b, c, 0, 1], bybuf)
            pltpu.sync_copy(iv_hbm.at[b, c, 1, 0], dxbuf)
            pltpu.sync_copy(iv_hbm.at[b, c, 1, 1], dybuf)
            lax.fori_loop(0, GROUPS, index_body, bc * HW)

            cp_b = pltpu.async_copy(pred_hbm.at[bidx], bvals, sem_b)
            cp_d = pltpu.async_copy(pred_hbm.at[didx], dvals, sem_d)
            cp_b.wait()
            cp_d.wait()

            def acc_body(j, a):
                bv = bvals[pl.ds(j * L, L)]
                dv = dvals[pl.ds(j * L, L)]
                d = bv - dv
                d2 = d * d
                good = (j * L + iota) < t_vec
                return a + jnp.where(good, 1.0 - d2, d2)

            acc = lax.fori_loop(0, GROUPS, acc_body, acc)

    accbuf[...] = acc
    pltpu.sync_copy(accbuf, out_hbm.at[wid])


@jax.jit
def _loss(pred_flat, iv0_t, iv1_t, nc0, nc1):
    mesh = plsc.VectorSubcoreMesh(core_axis_name="c", subcore_axis_name="s")
    run = pl.kernel(
        _loss_body,
        out_type=jax.ShapeDtypeStruct((NW, L), jnp.float32),
        mesh=mesh,
        compiler_params=pltpu.CompilerParams(needs_layout_passes=False),
        scratch_types=[
            pltpu.VMEM((CH,), jnp.int32),
            pltpu.VMEM((CH,), jnp.int32),
            pltpu.VMEM((CH,), jnp.int32),
            pltpu.VMEM((CH,), jnp.int32),
            pltpu.VMEM((CH,), jnp.int32),
            pltpu.VMEM((CH,), jnp.int32),
            pltpu.VMEM((CH,), jnp.float32),
            pltpu.VMEM((CH,), jnp.float32),
            pltpu.VMEM((L,), jnp.int32),
            pltpu.VMEM((L,), jnp.int32),
            pltpu.VMEM((L,), jnp.float32),
            pltpu.SemaphoreType.DMA,
            pltpu.SemaphoreType.DMA,
        ],
    )
    partials = run(pred_flat, iv0_t, iv1_t, nc0, nc1)
    return jnp.sum(partials)


def kernel(prediction, intervals_comp_0, intervals_comp_1,
           good_intervals_0, good_intervals_1):
    pred_flat = (prediction.reshape(B, C, H // 8, 8, W // 128, 128)
                 .transpose(0, 1, 2, 4, 3, 5).reshape(-1))
    iv0_t = intervals_comp_0.transpose(0, 1, 3, 4, 2)
    iv1_t = intervals_comp_1.transpose(0, 1, 3, 4, 2)
    return _loss(pred_flat, iv0_t, iv1_t,
                 good_intervals_0, good_intervals_1)


# speedup vs baseline: 33.5642x; 1.2754x over previous
import functools

import jax
import jax.numpy as jnp
from jax import lax
from jax.experimental import pallas as pl
from jax.experimental.pallas import tpu as pltpu
from jax.experimental.pallas import tpu_sc as plsc

B, C, H, W, I = 8, 16, 512, 512, 4096
NC, NS, L = 2, 16, 16
NW = NC * NS
BC = B * C
BC_PER_W = BC // NW
HW = H * W
CH = I
UNROLL = 4
GROUPS = CH // (L * UNROLL)
NCHUNK = 2 * BC_PER_W


def _phys_addr(pbase, x, y):
    xhi = lax.shift_right_logical(x, 3)
    xlo = lax.bitwise_and(x, 7)
    yhi = lax.shift_right_logical(y, 7)
    ylo = lax.bitwise_and(y, 127)
    return pbase + xhi * 4096 + yhi * 1024 + xlo * 128 + ylo


def _loss_body(pred_hbm, iv0_hbm, iv1_hbm, nc0_hbm, nc1_hbm, out_hbm,
               bx0, by0, dx0, dy0, bx1, by1, dx1, dy1,
               bidx0, didx0, bidx1, didx1,
               bvals0, dvals0, bvals1, dvals1,
               nc0buf, nc1buf, accbuf,
               semf0, semf1, semg0, semg1):
    cid = lax.axis_index("c")
    sid = lax.axis_index("s")
    wid = sid * NC + cid

    iota = lax.iota(jnp.int32, L)

    pltpu.sync_copy(nc0_hbm, nc0buf)
    pltpu.sync_copy(nc1_hbm, nc1buf)

    fields = ((bx0, by0, dx0, dy0), (bx1, by1, dx1, dy1))
    idxs = ((bidx0, didx0), (bidx1, didx1))
    vals = ((bvals0, dvals0), (bvals1, dvals1))
    semf = (semf0, semf1)
    semg = (semg0, semg1)

    ivs = (iv0_hbm, iv0_hbm, iv0_hbm, iv0_hbm,
           iv1_hbm, iv1_hbm, iv1_hbm, iv1_hbm)
    ncbufs = (nc0buf, nc0buf, nc0buf, nc0buf,
              nc1buf, nc1buf, nc1buf, nc1buf)

    def chunk_bc(t):
        return wid * BC_PER_W + (t % BC_PER_W)

    def start_fields(t):
        bc = chunk_bc(t)
        b = lax.div(bc, C)
        c = lax.rem(bc, C)
        iv = ivs[t]
        fb = fields[t % 2]
        sem = semf[t % 2]
        return [pltpu.async_copy(iv.at[b, c, p, x], fb[2 * p + x], sem)
                for p in range(2) for x in range(2)]

    def build_indices(t):
        bc = chunk_bc(t)
        pbase = bc * HW
        fbx, fby, fdx, fdy = fields[t % 2]
        bidx, didx = idxs[t % 2]

        def body(j, carry):
            for u in range(UNROLL):
                sl = pl.ds((j * UNROLL + u) * L, L)
                bidx[sl] = _phys_addr(carry, fbx[sl], fby[sl])
                didx[sl] = _phys_addr(carry, fdx[sl], fdy[sl])
            return carry

        lax.fori_loop(0, GROUPS, body, pbase)

    def start_gathers(t):
        bidx, didx = idxs[t % 2]
        bv, dv = vals[t % 2]
        sem = semg[t % 2]
        return [pltpu.async_copy(pred_hbm.at[bidx], bv, sem),
                pltpu.async_copy(pred_hbm.at[didx], dv, sem)]

    def accumulate(t, acc):
        bc = chunk_bc(t)
        c = lax.rem(bc, C)
        ncbuf = ncbufs[t]
        t_vec = plsc.load_gather(ncbuf, [jnp.full((L,), c, jnp.int32)])
        t_vec = jnp.minimum(t_vec, I)
        bv, dv = vals[t % 2]

        def body(j, a):
            for u in range(UNROLL):
                g = j * UNROLL + u
                sl = pl.ds(g * L, L)
                d = bv[sl] - dv[sl]
                d2 = d * d
                good = (g * L + iota) < t_vec
                a = a + jnp.where(good, 1.0 - d2, d2)
            return a

        return lax.fori_loop(0, GROUPS, body, acc)

    acc = jnp.zeros((L,), jnp.float32)
    f_cps = start_fields(0)
    g_cps = None
    for t in range(NCHUNK):
        for cp in f_cps:
            cp.wait()
        if t + 1 < NCHUNK:
            f_cps = start_fields(t + 1)
        build_indices(t)
        new_g = start_gathers(t)
        if g_cps is not None:
            for cp in g_cps:
                cp.wait()
            acc = accumulate(t - 1, acc)
        g_cps = new_g
    for cp in g_cps:
        cp.wait()
    acc = accumulate(NCHUNK - 1, acc)

    accbuf[...] = acc
    pltpu.sync_copy(accbuf, out_hbm.at[wid])


@jax.jit
def _loss(pred_flat, iv0_t, iv1_t, nc0, nc1):
    mesh = plsc.VectorSubcoreMesh(core_axis_name="c", subcore_axis_name="s")
    run = pl.kernel(
        _loss_body,
        out_type=jax.ShapeDtypeStruct((NW, L), jnp.float32),
        mesh=mesh,
        compiler_params=pltpu.CompilerParams(needs_layout_passes=False),
        scratch_types=(
            [pltpu.VMEM((CH,), jnp.int32) for _ in range(8)]
            + [pltpu.VMEM((CH,), jnp.int32) for _ in range(4)]
            + [pltpu.VMEM((CH,), jnp.float32) for _ in range(4)]
            + [pltpu.VMEM((L,), jnp.int32),
               pltpu.VMEM((L,), jnp.int32),
               pltpu.VMEM((L,), jnp.float32),
               pltpu.SemaphoreType.DMA,
               pltpu.SemaphoreType.DMA,
               pltpu.SemaphoreType.DMA,
               pltpu.SemaphoreType.DMA]
        ),
    )
    partials = run(pred_flat, iv0_t, iv1_t, nc0, nc1)
    return jnp.sum(partials)


def kernel(prediction, intervals_comp_0, intervals_comp_1,
           good_intervals_0, good_intervals_1):
    pred_flat = (prediction.reshape(B, C, H // 8, 8, W // 128, 128)
                 .transpose(0, 1, 2, 4, 3, 5).reshape(-1))
    iv0_t = intervals_comp_0.transpose(0, 1, 3, 4, 2)
    iv1_t = intervals_comp_1.transpose(0, 1, 3, 4, 2)
    return _loss(pred_flat, iv0_t, iv1_t,
                 good_intervals_0, good_intervals_1)
